# baseline (speedup 1.0000x reference)
import jax
import jax.numpy as jnp
from jax import lax
from jax.experimental import pallas as pl
from jax.experimental.pallas import tpu as pltpu
from jax.experimental.pallas import tpu_sc as plsc

VOCAB = 1000000
EMB = 64
SEQ = 200
BATCH = 1024

NC = 2
NS = 16
L = 16
NW = NC * NS
ROWS = SEQ * BATCH
RPW = ROWS // NW
CHUNK = SEQ
NCHUNK = RPW // CHUNK
G = 100
NG = CHUNK // G


def _emb_body(idx_hbm, table_hbm, pos_hbm, out_hbm,
              iv0, iv1, r0v, r1v, o0v, o1v, pos_v, g0, g1, w0, w1):
    wid = lax.axis_index("s") * NC + lax.axis_index("c")
    base = wid * (RPW // SEQ)
    ivs = [iv0, iv1]
    rows = [r0v, r1v]
    outs = [o0v, o1v]
    gsems = [g0, g1]
    osems = [w0, w1]
    pltpu.sync_copy(pos_hbm, pos_v)

    def stage_idx(grp):
        r0 = pl.multiple_of(wid * (RPW // G) + grp * 8, 8)
        pltpu.sync_copy(idx_hbm.at[pl.ds(r0, 8)], ivs[grp % 2])

    def gathers(c, p):
        return [
            pltpu.make_async_copy(
                table_hbm.at[ivs[(c // 4) % 2].at[2 * (c % 4) + k]],
                rows[p].at[pl.ds(G * k, G)],
                gsems[p],
            )
            for k in range(NG)
        ]

    def out_copy(c, p):
        return pltpu.make_async_copy(outs[p], out_hbm.at[pl.ds(base + c, 1)], osems[p])

    stage_idx(0)
    for cp in gathers(0, 0):
        cp.start()
    for c in range(NCHUNK):
        p = c % 2
        if c + 1 < NCHUNK:
            if (c + 1) % 4 == 0:
                stage_idx((c + 1) // 4)
            for cp in gathers(c + 1, (c + 1) % 2):
                cp.start()
        for cp in gathers(c, p):
            cp.wait()
        if c >= 2:
            out_copy(c - 2, p).wait()

        def body(t, carry):
            for j in range(EMB // L):
                outs[p][0, t, pl.ds(j * L, L)] = (
                    rows[p][t, pl.ds(j * L, L)] + pos_v[t, pl.ds(j * L, L)]
                )
            return carry

        lax.fori_loop(0, SEQ, body, 0)
        out_copy(c, p).start()
    out_copy(NCHUNK - 2, 0).wait()
    out_copy(NCHUNK - 1, 1).wait()


W = 32768
NTP = (VOCAB + W - 1) // W


def _tp_body(x_ref, o_ref):
    o_ref[:, 0:64] = x_ref[...].T


def _tc_relayout(wtT):
    return pl.pallas_call(
        _tp_body,
        grid=(NTP,),
        in_specs=[pl.BlockSpec((EMB, W), lambda i: (0, i))],
        out_specs=pl.BlockSpec((W, 128), lambda i: (i, 0)),
        out_shape=jax.ShapeDtypeStruct((VOCAB, 128), jnp.float32),
    )(wtT)


def kernel(sentence, word_table, pos_table):
    wt128 = _tc_relayout(jnp.transpose(word_table, (1, 0)))
    idx = jnp.transpose(sentence, (1, 0)).reshape(ROWS // G, G)
    pos = jnp.pad(
        lax.slice_in_dim(pos_table, 1, SEQ + 1, axis=0), ((0, 0), (0, 128 - EMB))
    )
    mesh = plsc.VectorSubcoreMesh(core_axis_name="c", subcore_axis_name="s")
    out = pl.kernel(
        _emb_body,
        out_type=jax.ShapeDtypeStruct((BATCH, SEQ, EMB), jnp.float32),
        mesh=mesh,
        compiler_params=pltpu.CompilerParams(
            use_tc_tiling_on_sc=True, needs_layout_passes=False
        ),
        scratch_types=[
            pltpu.VMEM((8, G), jnp.int32),
            pltpu.VMEM((8, G), jnp.int32),
            pltpu.VMEM((CHUNK, 128), jnp.float32),
            pltpu.VMEM((CHUNK, 128), jnp.float32),
            pltpu.VMEM((1, SEQ, EMB), jnp.float32),
            pltpu.VMEM((1, SEQ, EMB), jnp.float32),
            pltpu.VMEM((SEQ, 128), jnp.float32),
            pltpu.SemaphoreType.DMA,
            pltpu.SemaphoreType.DMA,
            pltpu.SemaphoreType.DMA,
            pltpu.SemaphoreType.DMA,
        ],
    )(idx, wt128, pos)
    return out
